# baseline (speedup 1.0000x reference)
import numpy as np
import jax
import jax.numpy as jnp
from jax.experimental import pallas as pl
from jax.experimental.pallas import tpu as pltpu

_H = 512
_W = 512
_HW = _H * _W
_D = 64
_N = 2
_ROWS = 32
_K = _H // _ROWS
_GAMMA = 0.9

_NFG, _NSEL, _NC0, _NC1, _FISH = 0, 1, 2, 3, 4
_S0 = 5


def _rank_maps(n: int) -> np.ndarray:
    rng = np.random.RandomState(0)
    out = np.empty((n, _HW), np.int32)
    for i in range(n):
        perm = rng.permutation(_HW)
        out[i, perm] = np.arange(_HW, dtype=np.int32)
    return out.reshape(n, _H, _W)


_RANKS = _rank_maps(_N)


def _fused_kernel(tgt_ref, ft_ref, rank_ref, activ_ref, pred_ref,
                  out_ref, acc_ref, cvec_ref, sm_ref):
    k = pl.program_id(1)
    is_p1 = k < _K

    @pl.when(k == 0)
    def _():
        nfg = jnp.sum((tgt_ref[0] == 1).astype(jnp.float32))
        sel_full = (rank_ref[0].astype(jnp.float32) < nfg) & (ft_ref[0] == 0)
        sm_ref[_NFG] = nfg
        sm_ref[_NSEL] = jnp.sum(sel_full.astype(jnp.float32))

    nfg = sm_ref[_NFG]
    r0 = jnp.where(is_p1, k, k - _K) * _ROWS
    tgt = tgt_ref[0, pl.ds(r0, _ROWS), :]
    ft = ft_ref[0, pl.ds(r0, _ROWS), :]
    rank = rank_ref[0, pl.ds(r0, _ROWS), :]
    fg_b = tgt == 1
    sel_b = (rank.astype(jnp.float32) < nfg) & (ft == 0)
    a = activ_ref[0]

    @pl.when(is_p1)
    def _():
        sel = sel_b.astype(jnp.float32)
        fg = fg_b.astype(jnp.float32)
        p_sel = jnp.sum(a * sel[None, :, :], axis=1)
        p_fg = jnp.sum(a * fg[None, :, :], axis=1)
        part = jnp.stack([p_sel, p_fg], axis=0)

        @pl.when(k == 0)
        def _():
            acc_ref[...] = part

        @pl.when(k > 0)
        def _():
            acc_ref[...] = acc_ref[...] + part

    @pl.when(k == _K)
    def _():
        c0 = jnp.sum(acc_ref[0], axis=1, keepdims=True) / sm_ref[_NSEL]
        c1 = jnp.sum(acc_ref[1], axis=1, keepdims=True) / sm_ref[_NFG]
        nc0 = jnp.sqrt(jnp.sum(c0 * c0))
        nc1 = jnp.sqrt(jnp.sum(c1 * c1))
        sm_ref[_NC0] = nc0
        sm_ref[_NC1] = nc1
        sm_ref[_FISH] = jnp.sum(c0 * c1) / (nc0 * nc1)
        cvec_ref[0] = jnp.broadcast_to(c0, (_D, 128))
        cvec_ref[1] = jnp.broadcast_to(c1, (_D, 128))
        sm_ref[_S0 + 0] = 0.0
        sm_ref[_S0 + 1] = 0.0
        sm_ref[_S0 + 2] = 0.0
        sm_ref[_S0 + 3] = 0.0

    @pl.when(jnp.logical_not(is_p1))
    def _():
        c0 = jnp.reshape(cvec_ref[0, :, 0:1], (_D, 1, 1))
        c1 = jnp.reshape(cvec_ref[1, :, 0:1], (_D, 1, 1))
        dot0 = jnp.sum(a * c0, axis=0)
        dot1 = jnp.sum(a * c1, axis=0)
        norm_p = jnp.sqrt(jnp.sum(a * a, axis=0))
        sim0 = dot0 / (sm_ref[_NC0] * norm_p)
        sim1 = dot1 / (sm_ref[_NC1] * norm_p)
        pseudo = jnp.where(sim1 > _GAMMA, 1,
                           jnp.where(sim0 > _GAMMA, 0, 2))

        w_valid = ((tgt == 0) & (~sel_b)
                   & (pseudo != 2)).astype(jnp.float32)
        sup = (fg_b | sel_b).astype(jnp.float32)

        p0 = pred_ref[0, 0]
        p1 = pred_ref[0, 1]
        m = jnp.maximum(p0, p1)
        lse = m + jnp.log(jnp.exp(p0 - m) + jnp.exp(p1 - m))
        ce0 = lse - p0
        ce1 = lse - p1
        ce_sup = jnp.where(tgt == 1, ce1, ce0)
        ce_self = jnp.where(pseudo >= 1, ce1, ce0)

        sm_ref[_S0 + 0] += jnp.sum(ce_sup * sup)
        sm_ref[_S0 + 1] += jnp.sum(sup)
        sm_ref[_S0 + 2] += jnp.sum(ce_self * w_valid)
        sm_ref[_S0 + 3] += jnp.sum(w_valid)

    @pl.when(k == 2 * _K - 1)
    def _():
        out_ref[0, 0] = jnp.full((8, 128), sm_ref[_S0 + 0], jnp.float32)
        out_ref[0, 1] = jnp.full((8, 128), sm_ref[_S0 + 1], jnp.float32)
        out_ref[0, 2] = jnp.full((8, 128), sm_ref[_S0 + 2], jnp.float32)
        out_ref[0, 3] = jnp.full((8, 128), sm_ref[_S0 + 3], jnp.float32)
        out_ref[0, 4] = jnp.full((8, 128), sm_ref[_FISH], jnp.float32)


def _img_spec():
    return pl.BlockSpec((1, _H, _W), lambda i, k: (i, 0, 0))


def kernel(activ_last_layer, predict, target, T1, T2, t, lam, full_target):
    n, d, h, w = activ_last_layer.shape
    grid = (n, 2 * _K)

    def _activ_map(i, k):
        return (i, 0, jnp.where(k < _K, k, k - _K), 0)

    def _pred_map(i, k):
        return (i, 0, jnp.maximum(k - _K, 0), 0)

    sums = pl.pallas_call(
        _fused_kernel,
        grid=grid,
        in_specs=[
            _img_spec(),
            _img_spec(),
            _img_spec(),
            pl.BlockSpec((1, _D, _ROWS, _W), _activ_map),
            pl.BlockSpec((1, 2, _ROWS, _W), _pred_map),
        ],
        out_specs=pl.BlockSpec((1, 5, 8, 128), lambda i, k: (i, 0, 0, 0)),
        out_shape=jax.ShapeDtypeStruct((n, 5, 8, 128), jnp.float32),
        scratch_shapes=[
            pltpu.VMEM((2, _D, _W), jnp.float32),
            pltpu.VMEM((2, _D, 128), jnp.float32),
            pltpu.SMEM((9,), jnp.float32),
        ],
    )(target, full_target, _RANKS, activ_last_layer, predict)

    s = sums[:, :, 0, 0]
    loss_sup = jnp.sum(s[:, 0] / s[:, 1])
    loss_self_sup = jnp.sum(s[:, 2] / s[:, 3])
    fisher_num = jnp.sum(s[:, 4])

    nan_flag = jnp.isnan(loss_self_sup) | jnp.isnan(fisher_num)
    alpha = jnp.where(t < T1, jnp.float32(0.0),
                      jnp.where(t < T2, (t - T1) * lam / (T2 - T1), lam))
    loss = loss_sup + alpha * loss_self_sup
    out = jnp.where(nan_flag, loss_sup, loss)
    return jnp.where(t < T1, loss_sup, out)


# speedup vs baseline: 1.1079x; 1.1079x over previous
import numpy as np
import jax
import jax.numpy as jnp
from jax.experimental import pallas as pl
from jax.experimental.pallas import tpu as pltpu

_H = 512
_W = 512
_HW = _H * _W
_D = 64
_N = 2
_ROWS = 64
_DH = _D // 2
_K = _H // _ROWS
_GAMMA = 0.9

_NFG, _NSEL, _NC0, _NC1, _FISH = 0, 1, 2, 3, 4
_S0 = 5


def _rank_maps(n: int) -> np.ndarray:
    rng = np.random.RandomState(0)
    out = np.empty((n, _HW), np.int32)
    for i in range(n):
        perm = rng.permutation(_HW)
        out[i, perm] = np.arange(_HW, dtype=np.int32)
    return out.reshape(n, _H, _W)


_RANKS = _rank_maps(_N)


def _fused_kernel(tgt_ref, ft_ref, rank_ref, alo_ref, ahi_ref, pred_ref,
                  out_ref, acc_ref, cvec_ref, sm_ref):
    k = pl.program_id(1)
    is_p1 = k < _K

    @pl.when(k == 0)
    def _():
        nfg = jnp.sum((tgt_ref[0] == 1).astype(jnp.float32))
        sel_full = (rank_ref[0].astype(jnp.float32) < nfg) & (ft_ref[0] == 0)
        sm_ref[_NFG] = nfg
        sm_ref[_NSEL] = jnp.sum(sel_full.astype(jnp.float32))

    nfg = sm_ref[_NFG]
    r0 = jnp.where(is_p1, k, k - _K) * _ROWS
    tgt = tgt_ref[0, pl.ds(r0, _ROWS), :]
    ft = ft_ref[0, pl.ds(r0, _ROWS), :]
    rank = rank_ref[0, pl.ds(r0, _ROWS), :]
    fg_b = tgt == 1
    sel_b = (rank.astype(jnp.float32) < nfg) & (ft == 0)
    alo = alo_ref[0]
    ahi = ahi_ref[0]

    @pl.when(is_p1)
    def _():
        sel = sel_b.astype(jnp.float32)
        fg = fg_b.astype(jnp.float32)
        p_sel = jnp.concatenate(
            [jnp.sum(alo * sel[None, :, :], axis=1),
             jnp.sum(ahi * sel[None, :, :], axis=1)], axis=0)
        p_fg = jnp.concatenate(
            [jnp.sum(alo * fg[None, :, :], axis=1),
             jnp.sum(ahi * fg[None, :, :], axis=1)], axis=0)
        part = jnp.stack([p_sel, p_fg], axis=0)

        @pl.when(k == 0)
        def _():
            acc_ref[...] = part

        @pl.when(k > 0)
        def _():
            acc_ref[...] = acc_ref[...] + part

    @pl.when(k == _K)
    def _():
        c0 = jnp.sum(acc_ref[0], axis=1, keepdims=True) / sm_ref[_NSEL]
        c1 = jnp.sum(acc_ref[1], axis=1, keepdims=True) / sm_ref[_NFG]
        nc0 = jnp.sqrt(jnp.sum(c0 * c0))
        nc1 = jnp.sqrt(jnp.sum(c1 * c1))
        sm_ref[_NC0] = nc0
        sm_ref[_NC1] = nc1
        sm_ref[_FISH] = jnp.sum(c0 * c1) / (nc0 * nc1)
        cvec_ref[0] = jnp.broadcast_to(c0, (_D, 128))
        cvec_ref[1] = jnp.broadcast_to(c1, (_D, 128))
        sm_ref[_S0 + 0] = 0.0
        sm_ref[_S0 + 1] = 0.0
        sm_ref[_S0 + 2] = 0.0
        sm_ref[_S0 + 3] = 0.0

    @pl.when(jnp.logical_not(is_p1))
    def _():
        c0lo = jnp.reshape(cvec_ref[0, 0:_DH, 0:1], (_DH, 1, 1))
        c0hi = jnp.reshape(cvec_ref[0, _DH:_D, 0:1], (_DH, 1, 1))
        c1lo = jnp.reshape(cvec_ref[1, 0:_DH, 0:1], (_DH, 1, 1))
        c1hi = jnp.reshape(cvec_ref[1, _DH:_D, 0:1], (_DH, 1, 1))
        dot0 = (jnp.sum(alo * c0lo, axis=0)
                + jnp.sum(ahi * c0hi, axis=0))
        dot1 = (jnp.sum(alo * c1lo, axis=0)
                + jnp.sum(ahi * c1hi, axis=0))
        norm_p = jnp.sqrt(jnp.sum(alo * alo, axis=0)
                          + jnp.sum(ahi * ahi, axis=0))
        sim0 = dot0 / (sm_ref[_NC0] * norm_p)
        sim1 = dot1 / (sm_ref[_NC1] * norm_p)
        pseudo = jnp.where(sim1 > _GAMMA, 1,
                           jnp.where(sim0 > _GAMMA, 0, 2))

        w_valid = ((tgt == 0) & (~sel_b)
                   & (pseudo != 2)).astype(jnp.float32)
        sup = (fg_b | sel_b).astype(jnp.float32)

        p0 = pred_ref[0, 0]
        p1 = pred_ref[0, 1]
        m = jnp.maximum(p0, p1)
        lse = m + jnp.log(jnp.exp(p0 - m) + jnp.exp(p1 - m))
        ce0 = lse - p0
        ce1 = lse - p1
        ce_sup = jnp.where(tgt == 1, ce1, ce0)
        ce_self = jnp.where(pseudo >= 1, ce1, ce0)

        sm_ref[_S0 + 0] += jnp.sum(ce_sup * sup)
        sm_ref[_S0 + 1] += jnp.sum(sup)
        sm_ref[_S0 + 2] += jnp.sum(ce_self * w_valid)
        sm_ref[_S0 + 3] += jnp.sum(w_valid)

    @pl.when(k == 2 * _K - 1)
    def _():
        out_ref[0, 0] = jnp.full((8, 128), sm_ref[_S0 + 0], jnp.float32)
        out_ref[0, 1] = jnp.full((8, 128), sm_ref[_S0 + 1], jnp.float32)
        out_ref[0, 2] = jnp.full((8, 128), sm_ref[_S0 + 2], jnp.float32)
        out_ref[0, 3] = jnp.full((8, 128), sm_ref[_S0 + 3], jnp.float32)
        out_ref[0, 4] = jnp.full((8, 128), sm_ref[_FISH], jnp.float32)


def _img_spec():
    return pl.BlockSpec((1, _H, _W), lambda i, k: (i, 0, 0))


def kernel(activ_last_layer, predict, target, T1, T2, t, lam, full_target):
    n, d, h, w = activ_last_layer.shape
    grid = (n, 2 * _K)

    def _activ_lo_map(i, k):
        return (i, 0, jnp.where(k < _K, k, k - _K), 0)

    def _activ_hi_map(i, k):
        return (i, 1, jnp.where(k < _K, k, k - _K), 0)

    def _pred_map(i, k):
        return (i, 0, jnp.maximum(k - _K, 0), 0)

    sums = pl.pallas_call(
        _fused_kernel,
        grid=grid,
        in_specs=[
            _img_spec(),
            _img_spec(),
            _img_spec(),
            pl.BlockSpec((1, _DH, _ROWS, _W), _activ_lo_map),
            pl.BlockSpec((1, _DH, _ROWS, _W), _activ_hi_map),
            pl.BlockSpec((1, 2, _ROWS, _W), _pred_map),
        ],
        out_specs=pl.BlockSpec((1, 5, 8, 128), lambda i, k: (i, 0, 0, 0)),
        out_shape=jax.ShapeDtypeStruct((n, 5, 8, 128), jnp.float32),
        scratch_shapes=[
            pltpu.VMEM((2, _D, _W), jnp.float32),
            pltpu.VMEM((2, _D, 128), jnp.float32),
            pltpu.SMEM((9,), jnp.float32),
        ],
    )(target, full_target, _RANKS, activ_last_layer, activ_last_layer,
      predict)

    s = sums[:, :, 0, 0]
    loss_sup = jnp.sum(s[:, 0] / s[:, 1])
    loss_self_sup = jnp.sum(s[:, 2] / s[:, 3])
    fisher_num = jnp.sum(s[:, 4])

    nan_flag = jnp.isnan(loss_self_sup) | jnp.isnan(fisher_num)
    alpha = jnp.where(t < T1, jnp.float32(0.0),
                      jnp.where(t < T2, (t - T1) * lam / (T2 - T1), lam))
    loss = loss_sup + alpha * loss_self_sup
    out = jnp.where(nan_flag, loss_sup, loss)
    return jnp.where(t < T1, loss_sup, out)
